# baseline (speedup 1.0000x reference)
import jax
import jax.numpy as jnp
from jax import lax
from jax.experimental import pallas as pl
from jax.experimental.pallas import tpu as pltpu
from jax.experimental.pallas import tpu_sc as plsc

N = 10000
E = 320000
D = 128

NC = 2
NS = 16
NW = NC * NS
EDGES_PER_W = E // NW
CHUNK = 80
NCH = 128
GRP = 8
NGP = NCH // GRP
N_PAD = 10240
ROWS_PER_TILE = N_PAD // NS
ZBLK = 128
DEG_ZBLK = 1024


def _make_sc_kernel():
    mesh = plsc.VectorSubcoreMesh(core_axis_name="c", subcore_axis_name="s",
                                  num_cores=NC, num_subcores=NS)

    def body(x_hbm, src_hbm, dst_hbm, z2_hbm, z1_hbm, ones_hbm,
             agg_hbm, deg_hbm,
             sb0, sb1, db0, db1, rows0, rows1, ones_v, agg_sh, deg_sh,
             g0, g1, is0, is1, id0, id1, qsem):
        c = lax.axis_index("c")
        s = lax.axis_index("s")
        wid = s * NC + c
        sb, db = (sb0, sb1), (db0, db1)
        rows = (rows0, rows1)
        gsem, isem, dsem = (g0, g1), (is0, is1), (id0, id1)

        pltpu.sync_copy(ones_hbm, ones_v)

        row0 = s * ROWS_PER_TILE
        for k in range(ROWS_PER_TILE // ZBLK):
            pltpu.sync_copy(z2_hbm, agg_sh.at[pl.ds(row0 + k * ZBLK, ZBLK)])

        @pl.when(s == 0)
        def _zero_deg():
            for k in range(N_PAD // DEG_ZBLK):
                pltpu.sync_copy(z1_hbm, deg_sh.at[pl.ds(k * DEG_ZBLK,
                                                        DEG_ZBLK)])

        def idx_prefetch(grp, slot):
            pltpu.async_copy(src_hbm.at[wid, pl.ds(grp * GRP, GRP)],
                             sb[slot], isem[slot])
            pltpu.async_copy(dst_hbm.at[wid, pl.ds(grp * GRP, GRP)],
                             db[slot], dsem[slot])

        def idx_wait(slot):
            pltpu.make_async_copy(src_hbm.at[0, pl.ds(0, GRP)],
                                  sb[slot], isem[slot]).wait()
            pltpu.make_async_copy(dst_hbm.at[0, pl.ds(0, GRP)],
                                  db[slot], dsem[slot]).wait()

        def deg_start(p, b):
            pltpu.async_copy(ones_v, deg_sh.at[db[p].at[b]], qsem, add=True)

        def deg_drain():
            for _ in range(GRP):
                pltpu.make_async_copy(ones_v, deg_sh.at[db0.at[0]],
                                      qsem).wait()

        def gather_start(slot, b, r):
            pltpu.async_copy(x_hbm.at[sb[slot].at[b]], rows[r], gsem[r])

        def gather_wait(r):
            pltpu.make_async_copy(x_hbm.at[sb0.at[0]], rows[r],
                                  gsem[r]).wait()

        idx_prefetch(0, 0)
        idx_wait(0)
        plsc.subcore_barrier()
        gather_start(0, 0, 0)
        gather_start(0, 1, 1)

        def run_group(g, p, skip_tail):
            @pl.when(g > 0)
            def _deg_drain():
                deg_drain()

            idx_prefetch(jnp.minimum(g + 1, NGP - 1), 1 - p)
            for b in range(GRP):
                r = b % 2
                gather_wait(r)
                pltpu.sync_copy(rows[r], agg_sh.at[db[p].at[b]], add=True)
                deg_start(p, b)
                if b == GRP - 2:
                    idx_wait(1 - p)
                if b < GRP - 2:
                    gather_start(p, b + 2, r)
                elif skip_tail is None:
                    gather_start(1 - p, b - (GRP - 2), r)
                else:
                    @pl.when(skip_tail)
                    def _tail():
                        gather_start(1 - p, b - (GRP - 2), r)

        @pl.loop(0, NGP // 2)
        def _pair(m):
            run_group(2 * m, 0, None)
            run_group(2 * m + 1, 1, m < NGP // 2 - 1)

        deg_drain()
        plsc.subcore_barrier()

        pltpu.sync_copy(agg_sh.at[pl.ds(row0, ROWS_PER_TILE)],
                        agg_hbm.at[c, pl.ds(row0, ROWS_PER_TILE)])

        @pl.when(s == 0)
        def _deg_out():
            pltpu.sync_copy(deg_sh, deg_hbm.at[c])

    return pl.kernel(
        body,
        out_type=(
            jax.ShapeDtypeStruct((NC, N_PAD, D), jnp.float32),
            jax.ShapeDtypeStruct((NC, N_PAD), jnp.float32),
        ),
        mesh=mesh,
        scratch_types=[
            pltpu.VMEM((GRP, CHUNK), jnp.int32),
            pltpu.VMEM((GRP, CHUNK), jnp.int32),
            pltpu.VMEM((GRP, CHUNK), jnp.int32),
            pltpu.VMEM((GRP, CHUNK), jnp.int32),
            pltpu.VMEM((CHUNK, D), jnp.float32),
            pltpu.VMEM((CHUNK, D), jnp.float32),
            pltpu.VMEM((CHUNK,), jnp.float32),
            pltpu.VMEM_SHARED((N_PAD, D), jnp.float32),
            pltpu.VMEM_SHARED((N_PAD,), jnp.float32),
            pltpu.SemaphoreType.DMA,
            pltpu.SemaphoreType.DMA,
            pltpu.SemaphoreType.DMA,
            pltpu.SemaphoreType.DMA,
            pltpu.SemaphoreType.DMA,
            pltpu.SemaphoreType.DMA,
            pltpu.SemaphoreType.DMA,
        ],
        name="sage_scatter_sc",
    )


_sc_kernel = _make_sc_kernel()

BLK = 2000


def _tc_base_body(x_ref, wr_ref, b_ref, o_ref):
    x = x_ref[...]
    o_ref[...] = 0.5 * (x + jnp.dot(x, wr_ref[...],
                                    preferred_element_type=jnp.float32)
                        + b_ref[...])


def _tc_final_body(base_ref, a0_ref, a1_ref, d0_ref, d1_ref, wl_ref, o_ref):
    deg = d0_ref[...] + d1_ref[...]
    inv = 0.5 / jnp.maximum(deg, 1.0)
    mean2 = (a0_ref[...] + a1_ref[...]) * inv
    o_ref[...] = base_ref[...] + jnp.dot(
        mean2, wl_ref[...], preferred_element_type=jnp.float32)


@jax.jit
def kernel(x, edge_index, W_l, b_l, W_r):
    pad = NCH * CHUNK - EDGES_PER_W
    pad_src = jnp.broadcast_to(jnp.arange(pad, dtype=jnp.int32), (NW, pad))
    pad_dst = pad_src + N
    src = jnp.concatenate(
        [edge_index[0].reshape(NW, EDGES_PER_W), pad_src],
        axis=1).reshape(NW, NCH, CHUNK)
    dst = jnp.concatenate(
        [edge_index[1].reshape(NW, EDGES_PER_W), pad_dst],
        axis=1).reshape(NW, NCH, CHUNK)
    zeros2d = jnp.zeros((ZBLK, D), jnp.float32)
    zeros1d = jnp.zeros((DEG_ZBLK,), jnp.float32)
    ones = jnp.ones((CHUNK,), jnp.float32)

    agg, deg = _sc_kernel(x, src, dst, zeros2d, zeros1d, ones)

    row_spec = pl.BlockSpec((BLK, D), lambda i: (i, 0))
    deg_spec = pl.BlockSpec((BLK, 1), lambda i: (i, 0))
    mat_spec = pl.BlockSpec((D, D), lambda i: (0, 0))
    bias_spec = pl.BlockSpec((1, D), lambda i: (0, 0))

    base = pl.pallas_call(
        _tc_base_body,
        grid=(N // BLK,),
        in_specs=[row_spec, mat_spec, bias_spec],
        out_specs=row_spec,
        out_shape=jax.ShapeDtypeStruct((N, D), jnp.float32),
    )(x, W_r.T, b_l[None, :])

    out = pl.pallas_call(
        _tc_final_body,
        grid=(N // BLK,),
        in_specs=[row_spec, row_spec, row_spec, deg_spec, deg_spec,
                  mat_spec],
        out_specs=row_spec,
        out_shape=jax.ShapeDtypeStruct((N, D), jnp.float32),
    )(base, agg[0], agg[1], deg[0, :, None], deg[1, :, None], W_l.T)
    return out


# speedup vs baseline: 1.1542x; 1.1542x over previous
import jax
import jax.numpy as jnp
from jax import lax
from jax.experimental import pallas as pl
from jax.experimental.pallas import tpu as pltpu
from jax.experimental.pallas import tpu_sc as plsc

N = 10000
E = 320000
D = 128

NC = 2
NS = 16
NW = NC * NS
EDGES_PER_W = E // NW
CHUNK = 64
NCH = 160
GRP = 8
NGP = NCH // GRP
NBUF = 4
N_PAD = 10240
ROWS_PER_TILE = N_PAD // NS
ZBLK = 128
DEG_ZBLK = 1024


def _make_sc_kernel():
    mesh = plsc.VectorSubcoreMesh(core_axis_name="c", subcore_axis_name="s",
                                  num_cores=NC, num_subcores=NS)

    def body(x_hbm, src_hbm, dst_hbm, z2_hbm, z1_hbm, ones_hbm,
             agg_hbm, deg_hbm,
             sb0, sb1, db0, db1, rows0, rows1, rows2, rows3, ones_v,
             agg_sh, deg_sh,
             g0, g1, g2, g3, is0, is1, id0, id1, qsem):
        c = lax.axis_index("c")
        s = lax.axis_index("s")
        wid = s * NC + c
        sb, db = (sb0, sb1), (db0, db1)
        rows = (rows0, rows1, rows2, rows3)
        gsem = (g0, g1, g2, g3)
        isem, dsem = (is0, is1), (id0, id1)

        pltpu.sync_copy(ones_hbm, ones_v)

        row0 = s * ROWS_PER_TILE
        for k in range(ROWS_PER_TILE // ZBLK):
            pltpu.sync_copy(z2_hbm, agg_sh.at[pl.ds(row0 + k * ZBLK, ZBLK)])

        @pl.when(s == 0)
        def _zero_deg():
            for k in range(N_PAD // DEG_ZBLK):
                pltpu.sync_copy(z1_hbm, deg_sh.at[pl.ds(k * DEG_ZBLK,
                                                        DEG_ZBLK)])

        def idx_prefetch(grp, slot):
            pltpu.async_copy(src_hbm.at[wid, pl.ds(grp * GRP, GRP)],
                             sb[slot], isem[slot])
            pltpu.async_copy(dst_hbm.at[wid, pl.ds(grp * GRP, GRP)],
                             db[slot], dsem[slot])

        def idx_wait(slot):
            pltpu.make_async_copy(src_hbm.at[0, pl.ds(0, GRP)],
                                  sb[slot], isem[slot]).wait()
            pltpu.make_async_copy(dst_hbm.at[0, pl.ds(0, GRP)],
                                  db[slot], dsem[slot]).wait()

        def deg_start(p, b):
            pltpu.async_copy(ones_v, deg_sh.at[db[p].at[b]], qsem, add=True)

        def deg_drain():
            for _ in range(GRP):
                pltpu.make_async_copy(ones_v, deg_sh.at[db0.at[0]],
                                      qsem).wait()

        def gather_start(slot, b, r):
            pltpu.async_copy(x_hbm.at[sb[slot].at[b]], rows[r], gsem[r])

        def gather_wait(r):
            pltpu.make_async_copy(x_hbm.at[sb0.at[0]], rows[r],
                                  gsem[r]).wait()

        idx_prefetch(0, 0)
        idx_wait(0)
        plsc.subcore_barrier()
        for b in range(NBUF):
            gather_start(0, b, b)

        def run_group(g, p, skip_tail):
            @pl.when(g > 0)
            def _deg_drain():
                deg_drain()

            idx_prefetch(jnp.minimum(g + 1, NGP - 1), 1 - p)
            for b in range(GRP):
                r = b % NBUF
                gather_wait(r)
                pltpu.sync_copy(rows[r], agg_sh.at[db[p].at[b]], add=True)
                deg_start(p, b)
                if b == GRP - NBUF:
                    idx_wait(1 - p)
                if b < GRP - NBUF:
                    gather_start(p, b + NBUF, r)
                elif skip_tail is None:
                    gather_start(1 - p, b - (GRP - NBUF), r)
                else:
                    @pl.when(skip_tail)
                    def _tail():
                        gather_start(1 - p, b - (GRP - NBUF), r)

        @pl.loop(0, NGP // 2)
        def _pair(m):
            run_group(2 * m, 0, None)
            run_group(2 * m + 1, 1, m < NGP // 2 - 1)

        deg_drain()
        plsc.subcore_barrier()

        pltpu.sync_copy(agg_sh.at[pl.ds(row0, ROWS_PER_TILE)],
                        agg_hbm.at[c, pl.ds(row0, ROWS_PER_TILE)])

        @pl.when(s == 0)
        def _deg_out():
            pltpu.sync_copy(deg_sh, deg_hbm.at[c])

    return pl.kernel(
        body,
        out_type=(
            jax.ShapeDtypeStruct((NC, N_PAD, D), jnp.float32),
            jax.ShapeDtypeStruct((NC, N_PAD), jnp.float32),
        ),
        mesh=mesh,
        scratch_types=[
            pltpu.VMEM((GRP, CHUNK), jnp.int32),
            pltpu.VMEM((GRP, CHUNK), jnp.int32),
            pltpu.VMEM((GRP, CHUNK), jnp.int32),
            pltpu.VMEM((GRP, CHUNK), jnp.int32),
            pltpu.VMEM((CHUNK, D), jnp.float32),
            pltpu.VMEM((CHUNK, D), jnp.float32),
            pltpu.VMEM((CHUNK, D), jnp.float32),
            pltpu.VMEM((CHUNK, D), jnp.float32),
            pltpu.VMEM((CHUNK,), jnp.float32),
            pltpu.VMEM_SHARED((N_PAD, D), jnp.float32),
            pltpu.VMEM_SHARED((N_PAD,), jnp.float32),
            pltpu.SemaphoreType.DMA,
            pltpu.SemaphoreType.DMA,
            pltpu.SemaphoreType.DMA,
            pltpu.SemaphoreType.DMA,
            pltpu.SemaphoreType.DMA,
            pltpu.SemaphoreType.DMA,
            pltpu.SemaphoreType.DMA,
            pltpu.SemaphoreType.DMA,
            pltpu.SemaphoreType.DMA,
        ],
        name="sage_scatter_sc",
    )


_sc_kernel = _make_sc_kernel()

BLK = 2000


def _tc_body(x_ref, a0_ref, a1_ref, d0_ref, d1_ref, wl_ref, wr_ref, b_ref,
             o_ref):
    deg = d0_ref[...] + d1_ref[...]
    inv = 1.0 / jnp.maximum(deg, 1.0)
    mean = (a0_ref[...] + a1_ref[...]) * inv
    x = x_ref[...]
    node_emb = (jnp.dot(mean, wl_ref[...], preferred_element_type=jnp.float32)
                + b_ref[...]
                + jnp.dot(x, wr_ref[...], preferred_element_type=jnp.float32))
    o_ref[...] = 0.5 * (x + node_emb)


@jax.jit
def kernel(x, edge_index, W_l, b_l, W_r):
    pad = NCH * CHUNK - EDGES_PER_W
    pad_src = jnp.broadcast_to(jnp.arange(pad, dtype=jnp.int32), (NW, pad))
    pad_dst = pad_src + N
    src = jnp.concatenate(
        [edge_index[0].reshape(NW, EDGES_PER_W), pad_src],
        axis=1).reshape(NW, NCH, CHUNK)
    dst = jnp.concatenate(
        [edge_index[1].reshape(NW, EDGES_PER_W), pad_dst],
        axis=1).reshape(NW, NCH, CHUNK)
    zeros2d = jnp.zeros((ZBLK, D), jnp.float32)
    zeros1d = jnp.zeros((DEG_ZBLK,), jnp.float32)
    ones = jnp.ones((CHUNK,), jnp.float32)

    agg, deg = _sc_kernel(x, src, dst, zeros2d, zeros1d, ones)

    row_spec = pl.BlockSpec((BLK, D), lambda i: (i, 0))
    deg_spec = pl.BlockSpec((BLK, 1), lambda i: (i, 0))
    mat_spec = pl.BlockSpec((D, D), lambda i: (0, 0))
    bias_spec = pl.BlockSpec((1, D), lambda i: (0, 0))

    out = pl.pallas_call(
        _tc_body,
        grid=(N // BLK,),
        in_specs=[row_spec, row_spec, row_spec, deg_spec, deg_spec,
                  mat_spec, mat_spec, bias_spec],
        out_specs=row_spec,
        out_shape=jax.ShapeDtypeStruct((N, D), jnp.float32),
    )(x, agg[0], agg[1], deg[0, :, None], deg[1, :, None],
      W_l.T, W_r.T, b_l[None, :])
    return out
